# baseline (speedup 1.0000x reference)
import functools

import jax
import jax.numpy as jnp
from jax import lax
from jax.experimental import pallas as pl
from jax.experimental.pallas import tpu as pltpu
from jax.experimental.pallas import tpu_sc as plsc

N_NODES = 100000
N_EDGES = 1600000
T_TASKS = 32
E_APP = 64

NC, NS = 2, 16
NW = NC * NS
ROW_E = 128
G_ROWS = 4
ROWS_PER_W = 392
N_CHUNKS = ROWS_PER_W // G_ROWS
E_PAD = NW * ROWS_PER_W * ROW_E
COLS = 8
N_ACC = N_NODES + 96
ROWS_PER_SUB = N_ACC // NS

BN = 2048
N_TC = 100352
GRID = N_TC // BN


def _sc_segment_sum(x5, src2d, dst2d, zeros):
    mesh = plsc.VectorSubcoreMesh(
        core_axis_name="c", subcore_axis_name="s", num_cores=NC, num_subcores=NS
    )

    @functools.partial(
        pl.kernel,
        out_type=jax.ShapeDtypeStruct((NC * N_ACC, COLS), jnp.float32),
        mesh=mesh,
        scratch_types=[
            pltpu.VMEM_SHARED((N_ACC, COLS), jnp.float32),
            pltpu.VMEM((2, G_ROWS, ROW_E), jnp.int32),
            pltpu.VMEM((2, G_ROWS, ROW_E), jnp.int32),
            pltpu.VMEM((2, G_ROWS * ROW_E, COLS), jnp.float32),
            pltpu.SemaphoreType.DMA,
            pltpu.SemaphoreType.DMA,
        ],
        compiler_params=pltpu.CompilerParams(use_tc_tiling_on_sc=False),
    )
    def body(x5_hbm, src_hbm, dst_hbm, z_hbm, out_hbm,
             acc, idx_s, idx_d, rows, sem0, sem1):
        c = lax.axis_index("c")
        s = lax.axis_index("s")
        wid = c * NS + s
        sems = (sem0, sem1)
        pltpu.sync_copy(
            z_hbm,
            acc.at[pl.ds(s * ROWS_PER_SUB, ROWS_PER_SUB)],
        )
        plsc.subcore_barrier()

        base = wid * ROWS_PER_W

        def stage_and_fire(chunk, b):
            r0 = base + chunk * G_ROWS
            pltpu.sync_copy(src_hbm.at[pl.ds(r0, G_ROWS)], idx_s.at[b])
            pltpu.sync_copy(dst_hbm.at[pl.ds(r0, G_ROWS)], idx_d.at[b])
            return [
                pltpu.async_copy(
                    x5_hbm.at[idx_s.at[b, j]],
                    rows.at[b, pl.ds(j * ROW_E, ROW_E)],
                    sems[b],
                )
                for j in range(G_ROWS)
            ]

        def scatter(b):
            for j in range(G_ROWS):
                pltpu.sync_copy(
                    rows.at[b, pl.ds(j * ROW_E, ROW_E)],
                    acc.at[idx_d.at[b, j]],
                    add=True,
                )

        def step(i, carry):
            k0 = i * 2
            d0 = stage_and_fire(k0, 0)
            d1 = stage_and_fire(k0 + 1, 1)
            for d in d0:
                d.wait()
            scatter(0)
            for d in d1:
                d.wait()
            scatter(1)
            return carry

        lax.fori_loop(0, N_CHUNKS // 2, step, 0)
        plsc.subcore_barrier()
        pltpu.sync_copy(
            acc.at[pl.ds(s * ROWS_PER_SUB, ROWS_PER_SUB)],
            out_hbm.at[pl.ds(c * N_ACC + s * ROWS_PER_SUB, ROWS_PER_SUB)],
        )

    return body(x5, src2d, dst2d, zeros)


def _tc_body(
    t_ref, appe_ref, appet_ref, axT_ref, wa1t_ref, wa2t_ref, wart_ref,
    wl1_ref, wl2_ref, want_ref, vatt_ref, wv_ref,
    rl_ref, act_ref, out_ref, stats, reqw, pooled,
):
    i = pl.program_id(0)

    @pl.when(i == 0)
    def _init():
        asrc_row = appe_ref[0:1, :]
        adst_col = appet_ref[:, 1:2]
        iota_t = lax.broadcasted_iota(jnp.int32, (T_TASKS, E_APP), 0)
        s_t = (iota_t == asrc_row).astype(jnp.float32)
        iota_l = lax.broadcasted_iota(jnp.int32, (E_APP, T_TASKS), 1)
        d2 = (iota_l == adst_col).astype(jnp.float32)
        deg = jnp.maximum(jnp.sum(d2, axis=0, keepdims=True), 1.0)
        msg1 = jnp.dot(axT_ref[...], s_t)
        agg1 = jnp.dot(msg1, d2) / deg
        h = jnp.maximum(jnp.dot(wa1t_ref[...], agg1), 0.0)
        msg2 = jnp.dot(h, s_t)
        agg2 = jnp.dot(msg2, d2) / deg
        h2 = jnp.dot(wa2t_ref[...], agg2)
        req_emb = jnp.mean(h2, axis=1, keepdims=True)
        rl = jnp.full((1, 1), rl_ref[0], jnp.float32)
        req_col = jnp.concatenate([req_emb, rl], axis=0)
        reqw_col = jnp.dot(wart_ref[...], req_col)
        reqw[...] = jnp.broadcast_to(reqw_col, (T_TASKS, 128))
        req_dot_wv = jnp.sum(req_col * wv_ref[32:36, 0:1], axis=0, keepdims=True)
        stats[0:1, :] = jnp.full((1, 128), -3e38, jnp.float32)
        stats[1:2, :] = jnp.zeros((1, 128), jnp.float32)
        stats[2:3, :] = jnp.zeros((1, 128), jnp.float32)
        stats[3:4, :] = jnp.zeros((1, 128), jnp.float32)
        stats[4:5, :] = jnp.broadcast_to(req_dot_wv, (1, 128))
        pooled[...] = jnp.zeros((T_TASKS, 128), jnp.float32)

    blk = t_ref[...]
    acc = blk[0] + blk[1]
    deg = jnp.maximum(acc[4:5, :], 1.0)
    agg = acc[0:4, :] / deg

    hc1 = jnp.maximum(jnp.dot(wl1_ref[...], agg), 0.0)
    hc2 = jnp.maximum(jnp.dot(wl2_ref[...], hc1), 0.0)
    h2 = hc2[0:64, :]
    c2 = hc2[64:96, :]
    att = jnp.tanh(jnp.dot(want_ref[...], h2) + reqw[:, 0:1])
    scores = jnp.sum(att * vatt_ref[...], axis=0, keepdims=True)

    gcol = i * BN + lax.broadcasted_iota(jnp.int32, (1, BN), 1)
    valid = gcol < N_NODES
    sm = jnp.where(valid, scores, -1e30)

    m_old = stats[0:1, 0:1]
    m_b = jnp.max(sm, axis=1, keepdims=True)
    m_new = jnp.maximum(m_old, m_b)
    alpha = jnp.exp(m_old - m_new)
    e = jnp.exp(sm - m_new)
    s1 = stats[1:2, 0:1] * alpha + jnp.sum(e, axis=1, keepdims=True)
    s2 = stats[2:3, 0:1] * alpha + jnp.sum(sm * e, axis=1, keepdims=True)
    a_b = jnp.sum(
        jnp.where(gcol == act_ref[0], scores, 0.0), axis=1, keepdims=True
    )
    s_act = stats[3:4, 0:1] + a_b
    stats[0:1, :] = jnp.broadcast_to(m_new, (1, 128))
    stats[1:2, :] = jnp.broadcast_to(s1, (1, 128))
    stats[2:3, :] = jnp.broadcast_to(s2, (1, 128))
    stats[3:4, :] = jnp.broadcast_to(s_act, (1, 128))

    c2m = jnp.where(valid, c2, 0.0)
    pooled[...] += jnp.broadcast_to(
        jnp.sum(c2m, axis=1, keepdims=True), (T_TASKS, 128)
    )

    @pl.when(i == GRID - 1)
    def _fin():
        m = stats[0:1, 0:1]
        s1f = stats[1:2, 0:1]
        s2f = stats[2:3, 0:1]
        sa = stats[3:4, 0:1]
        log_z = m + jnp.log(s1f)
        entropy = log_z - s2f / s1f
        alp = sa - log_z
        pooled_mean = pooled[:, 0:1] / float(N_NODES)
        sv = (
            jnp.sum(pooled_mean * wv_ref[0:32, 0:1], axis=0, keepdims=True)
            + stats[4:5, 0:1]
        )
        l = lax.broadcasted_iota(jnp.int32, (1, 128), 1)
        out_ref[...] = (
            jnp.where(l == 0, jnp.broadcast_to(alp, (1, 128)), 0.0)
            + jnp.where(l == 1, jnp.broadcast_to(sv, (1, 128)), 0.0)
            + jnp.where(l == 2, jnp.broadcast_to(entropy, (1, 128)), 0.0)
        )


def _tc_dense(t, app_edge, app_edge_t, app_xT, wa1t, wa2t, wart, wl1, wl2,
              want, vatt2, wv, rl, act):
    full = lambda shape: pl.BlockSpec(shape, lambda i: tuple(0 for _ in shape))
    return pl.pallas_call(
        _tc_body,
        grid=(GRID,),
        in_specs=[
            pl.BlockSpec((2, COLS, BN), lambda i: (0, 0, i)),
            full((2, E_APP)),
            full((E_APP, 2)),
            full((3, T_TASKS)),
            full((16, 3)),
            full((3, 16)),
            full((T_TASKS, 4)),
            full((128, 4)),
            full((96, 128)),
            full((T_TASKS, 64)),
            full((T_TASKS, 1)),
            full((36, 1)),
            pl.BlockSpec(memory_space=pltpu.SMEM),
            pl.BlockSpec(memory_space=pltpu.SMEM),
        ],
        out_specs=pl.BlockSpec((1, 128), lambda i: (0, 0)),
        out_shape=jax.ShapeDtypeStruct((1, 128), jnp.float32),
        scratch_shapes=[
            pltpu.VMEM((8, 128), jnp.float32),
            pltpu.VMEM((T_TASKS, 128), jnp.float32),
            pltpu.VMEM((T_TASKS, 128), jnp.float32),
        ],
    )(t, app_edge, app_edge_t, app_xT, wa1t, wa2t, wart, wl1, wl2, want,
      vatt2, wv, rl, act)


def kernel(x, edge_index, app_x, app_edge_index, requests_left, action,
           W_a1, W_a2, W_p1, W_p2, W_an, W_ar, v_att, W_c1, W_c2, W_v):
    f32 = jnp.float32
    x5 = jnp.concatenate(
        [x, jnp.ones((N_NODES, 1), f32), jnp.zeros((N_NODES, 3), f32)], axis=1
    )
    pad = E_PAD - N_EDGES
    src2d = jnp.pad(edge_index[0], (0, pad)).reshape(E_PAD // ROW_E, ROW_E)
    dst2d = jnp.pad(
        edge_index[1], (0, pad), constant_values=N_NODES
    ).reshape(E_PAD // ROW_E, ROW_E)
    zeros = jnp.zeros((ROWS_PER_SUB, COLS), f32)

    partials = _sc_segment_sum(x5, src2d, dst2d, zeros)
    t = partials.reshape(NC, N_ACC, COLS)[:, :N_NODES, :].transpose(0, 2, 1)
    t = jnp.pad(t, ((0, 0), (0, 0), (0, N_TC - N_NODES)))

    wl1 = jnp.concatenate([W_p1.T, W_c1.T], axis=0)
    wl2 = jnp.zeros((96, 128), f32)
    wl2 = wl2.at[0:64, 0:64].set(W_p2.T).at[64:96, 64:128].set(W_c2.T)

    out = _tc_dense(
        t,
        app_edge_index,
        app_edge_index.T,
        app_x.T,
        W_a1.T,
        W_a2.T,
        W_ar.T,
        wl1,
        wl2,
        W_an.T,
        v_att.reshape(T_TASKS, 1),
        W_v,
        requests_left,
        jnp.asarray(action, jnp.int32).reshape(1),
    )
    return out[0, 0:3]


# speedup vs baseline: 1.0612x; 1.0612x over previous
import functools

import jax
import jax.numpy as jnp
from jax import lax
from jax.experimental import pallas as pl
from jax.experimental.pallas import tpu as pltpu
from jax.experimental.pallas import tpu_sc as plsc

N_NODES = 100000
N_EDGES = 1600000
T_TASKS = 32
E_APP = 64

NC, NS = 2, 16
NW = NC * NS
ROW_E = 128
G_ROWS = 4
E_ROWS = N_EDGES // ROW_E
ROWS_PER_W = 392
W31_ROWS = E_ROWS - 31 * ROWS_PER_W
COLS = 8

BN = 2048
N_TC = 100352
GRID = N_TC // BN
N_ACC = N_TC
ROWS_PER_SUB = N_ACC // NS


def _sc_segment_sum(x5, e3, zeros):
    mesh = plsc.VectorSubcoreMesh(
        core_axis_name="c", subcore_axis_name="s", num_cores=NC, num_subcores=NS
    )

    @functools.partial(
        pl.kernel,
        out_type=(
            jax.ShapeDtypeStruct((N_ACC, COLS), jnp.float32),
            jax.ShapeDtypeStruct((N_ACC, COLS), jnp.float32),
        ),
        mesh=mesh,
        scratch_types=[
            pltpu.VMEM_SHARED((N_ACC, COLS), jnp.float32),
            pltpu.VMEM((2, G_ROWS, ROW_E), jnp.int32),
            pltpu.VMEM((2, G_ROWS, ROW_E), jnp.int32),
            pltpu.VMEM((2, G_ROWS * ROW_E, COLS), jnp.float32),
            pltpu.SemaphoreType.DMA,
            pltpu.SemaphoreType.DMA,
        ],
        compiler_params=pltpu.CompilerParams(use_tc_tiling_on_sc=False),
    )
    def body(x5_hbm, e3_hbm, z_hbm, out0_hbm, out1_hbm,
             acc, idx_s, idx_d, rows, sem0, sem1):
        c = lax.axis_index("c")
        s = lax.axis_index("s")
        wid = c * NS + s
        sems = (sem0, sem1)
        pltpu.sync_copy(
            z_hbm,
            acc.at[pl.ds(s * ROWS_PER_SUB, ROWS_PER_SUB)],
        )
        plsc.subcore_barrier()

        base = wid * ROWS_PER_W

        def stage_and_fire(chunk, b):
            r0 = base + chunk * G_ROWS
            pltpu.sync_copy(e3_hbm.at[0, pl.ds(r0, G_ROWS)], idx_s.at[b])
            pltpu.sync_copy(e3_hbm.at[1, pl.ds(r0, G_ROWS)], idx_d.at[b])
            return [
                pltpu.async_copy(
                    x5_hbm.at[idx_s.at[b, j]],
                    rows.at[b, pl.ds(j * ROW_E, ROW_E)],
                    sems[b],
                )
                for j in range(G_ROWS)
            ]

        def scatter(b):
            for j in range(G_ROWS):
                pltpu.sync_copy(
                    rows.at[b, pl.ds(j * ROW_E, ROW_E)],
                    acc.at[idx_d.at[b, j]],
                    add=True,
                )

        def step(i, carry):
            k0 = i * 2
            d0 = stage_and_fire(k0, 0)
            d1 = stage_and_fire(k0 + 1, 1)
            for d in d0:
                d.wait()
            scatter(0)
            for d in d1:
                d.wait()
            scatter(1)
            return carry

        npairs = jnp.where(wid == NW - 1, (W31_ROWS // G_ROWS) // 2,
                           (ROWS_PER_W // G_ROWS) // 2)
        lax.fori_loop(0, npairs, step, 0)

        @pl.when(wid == NW - 1)
        def _tail():
            d0 = stage_and_fire(W31_ROWS // G_ROWS - 1, 0)
            for d in d0:
                d.wait()
            scatter(0)

        plsc.subcore_barrier()

        @pl.when(c == 0)
        def _wb0():
            pltpu.sync_copy(
                acc.at[pl.ds(s * ROWS_PER_SUB, ROWS_PER_SUB)],
                out0_hbm.at[pl.ds(s * ROWS_PER_SUB, ROWS_PER_SUB)],
            )

        @pl.when(c == 1)
        def _wb1():
            pltpu.sync_copy(
                acc.at[pl.ds(s * ROWS_PER_SUB, ROWS_PER_SUB)],
                out1_hbm.at[pl.ds(s * ROWS_PER_SUB, ROWS_PER_SUB)],
            )

    return body(x5, e3, zeros)


def _tc_body(
    t_ref, appe_ref, appet_ref, axT_ref, wa1t_ref, wa2t_ref, wart_ref,
    wl1_ref, wl2_ref, want_ref, vatt_ref, wv_ref,
    rl_ref, act_ref, out_ref, stats, reqw, pooled,
):
    i = pl.program_id(0)

    @pl.when(i == 0)
    def _init():
        asrc_row = appe_ref[0:1, :]
        adst_col = appet_ref[:, 1:2]
        iota_t = lax.broadcasted_iota(jnp.int32, (T_TASKS, E_APP), 0)
        s_t = (iota_t == asrc_row).astype(jnp.float32)
        iota_l = lax.broadcasted_iota(jnp.int32, (E_APP, T_TASKS), 1)
        d2 = (iota_l == adst_col).astype(jnp.float32)
        deg = jnp.maximum(jnp.sum(d2, axis=0, keepdims=True), 1.0)
        msg1 = jnp.dot(axT_ref[...], s_t)
        agg1 = jnp.dot(msg1, d2) / deg
        h = jnp.maximum(jnp.dot(wa1t_ref[...], agg1), 0.0)
        msg2 = jnp.dot(h, s_t)
        agg2 = jnp.dot(msg2, d2) / deg
        h2 = jnp.dot(wa2t_ref[...], agg2)
        req_emb = jnp.mean(h2, axis=1, keepdims=True)
        rl = jnp.full((1, 1), rl_ref[0], jnp.float32)
        req_col = jnp.concatenate([req_emb, rl], axis=0)
        reqw_col = jnp.dot(wart_ref[...], req_col)
        reqw[...] = jnp.broadcast_to(reqw_col, (T_TASKS, 128))
        req_dot_wv = jnp.sum(req_col * wv_ref[32:36, 0:1], axis=0, keepdims=True)
        stats[0:1, :] = jnp.full((1, 128), -3e38, jnp.float32)
        stats[1:2, :] = jnp.zeros((1, 128), jnp.float32)
        stats[2:3, :] = jnp.zeros((1, 128), jnp.float32)
        stats[3:4, :] = jnp.zeros((1, 128), jnp.float32)
        stats[4:5, :] = jnp.broadcast_to(req_dot_wv, (1, 128))
        pooled[...] = jnp.zeros((T_TASKS, 128), jnp.float32)

    blk = t_ref[...]
    acc = blk[0] + blk[1]
    deg = jnp.maximum(acc[4:5, :], 1.0)
    agg = acc[0:4, :] / deg

    hc1 = jnp.maximum(jnp.dot(wl1_ref[...], agg), 0.0)
    hc2 = jnp.maximum(jnp.dot(wl2_ref[...], hc1), 0.0)
    h2 = hc2[0:64, :]
    c2 = hc2[64:96, :]
    att = jnp.tanh(jnp.dot(want_ref[...], h2) + reqw[:, 0:1])
    scores = jnp.sum(att * vatt_ref[...], axis=0, keepdims=True)

    gcol = i * BN + lax.broadcasted_iota(jnp.int32, (1, BN), 1)
    valid = gcol < N_NODES
    sm = jnp.where(valid, scores, -1e30)

    m_old = stats[0:1, 0:1]
    m_b = jnp.max(sm, axis=1, keepdims=True)
    m_new = jnp.maximum(m_old, m_b)
    alpha = jnp.exp(m_old - m_new)
    e = jnp.exp(sm - m_new)
    s1 = stats[1:2, 0:1] * alpha + jnp.sum(e, axis=1, keepdims=True)
    s2 = stats[2:3, 0:1] * alpha + jnp.sum(sm * e, axis=1, keepdims=True)
    a_b = jnp.sum(
        jnp.where(gcol == act_ref[0], scores, 0.0), axis=1, keepdims=True
    )
    s_act = stats[3:4, 0:1] + a_b
    stats[0:1, :] = jnp.broadcast_to(m_new, (1, 128))
    stats[1:2, :] = jnp.broadcast_to(s1, (1, 128))
    stats[2:3, :] = jnp.broadcast_to(s2, (1, 128))
    stats[3:4, :] = jnp.broadcast_to(s_act, (1, 128))

    c2m = jnp.where(valid, c2, 0.0)
    pooled[...] += jnp.broadcast_to(
        jnp.sum(c2m, axis=1, keepdims=True), (T_TASKS, 128)
    )

    @pl.when(i == GRID - 1)
    def _fin():
        m = stats[0:1, 0:1]
        s1f = stats[1:2, 0:1]
        s2f = stats[2:3, 0:1]
        sa = stats[3:4, 0:1]
        log_z = m + jnp.log(s1f)
        entropy = log_z - s2f / s1f
        alp = sa - log_z
        pooled_mean = pooled[:, 0:1] / float(N_NODES)
        sv = (
            jnp.sum(pooled_mean * wv_ref[0:32, 0:1], axis=0, keepdims=True)
            + stats[4:5, 0:1]
        )
        l = lax.broadcasted_iota(jnp.int32, (1, 128), 1)
        out_ref[...] = (
            jnp.where(l == 0, jnp.broadcast_to(alp, (1, 128)), 0.0)
            + jnp.where(l == 1, jnp.broadcast_to(sv, (1, 128)), 0.0)
            + jnp.where(l == 2, jnp.broadcast_to(entropy, (1, 128)), 0.0)
        )


def _tc_dense(t, app_edge, app_edge_t, app_xT, wa1t, wa2t, wart, wl1, wl2,
              want, vatt2, wv, rl, act):
    full = lambda shape: pl.BlockSpec(shape, lambda i: tuple(0 for _ in shape))
    return pl.pallas_call(
        _tc_body,
        grid=(GRID,),
        in_specs=[
            pl.BlockSpec((2, COLS, BN), lambda i: (0, 0, i)),
            full((2, E_APP)),
            full((E_APP, 2)),
            full((3, T_TASKS)),
            full((16, 3)),
            full((3, 16)),
            full((T_TASKS, 4)),
            full((128, 4)),
            full((96, 128)),
            full((T_TASKS, 64)),
            full((T_TASKS, 1)),
            full((36, 1)),
            pl.BlockSpec(memory_space=pltpu.SMEM),
            pl.BlockSpec(memory_space=pltpu.SMEM),
        ],
        out_specs=pl.BlockSpec((1, 128), lambda i: (0, 0)),
        out_shape=jax.ShapeDtypeStruct((1, 128), jnp.float32),
        scratch_shapes=[
            pltpu.VMEM((8, 128), jnp.float32),
            pltpu.VMEM((T_TASKS, 128), jnp.float32),
            pltpu.VMEM((T_TASKS, 128), jnp.float32),
        ],
    )(t, app_edge, app_edge_t, app_xT, wa1t, wa2t, wart, wl1, wl2, want,
      vatt2, wv, rl, act)


def kernel(x, edge_index, app_x, app_edge_index, requests_left, action,
           W_a1, W_a2, W_p1, W_p2, W_an, W_ar, v_att, W_c1, W_c2, W_v):
    f32 = jnp.float32
    x5 = jnp.concatenate(
        [x, jnp.ones((N_NODES, 1), f32), jnp.zeros((N_NODES, 3), f32)], axis=1
    )
    e3 = edge_index.reshape(2, E_ROWS, ROW_E)
    zeros = jnp.zeros((ROWS_PER_SUB, COLS), f32)

    p0, p1 = _sc_segment_sum(x5, e3, zeros)
    t = jnp.stack([p0, p1]).transpose(0, 2, 1)

    wl1 = jnp.concatenate([W_p1.T, W_c1.T], axis=0)
    wl2 = jnp.zeros((96, 128), f32)
    wl2 = wl2.at[0:64, 0:64].set(W_p2.T).at[64:96, 64:128].set(W_c2.T)

    out = _tc_dense(
        t,
        app_edge_index,
        app_edge_index.T,
        app_x.T,
        W_a1.T,
        W_a2.T,
        W_ar.T,
        wl1,
        wl2,
        W_an.T,
        v_att.reshape(T_TASKS, 1),
        W_v,
        requests_left,
        jnp.asarray(action, jnp.int32).reshape(1),
    )
    return out[0, 0:3]
